# speedup vs baseline: 8.8421x; 8.8421x over previous
import functools

import jax
import jax.numpy as jnp
from jax import lax
from jax.experimental import pallas as pl
from jax.experimental.pallas import tpu as pltpu
from jax.experimental.pallas import tpu_sc as plsc

_N = 10000
_E = 320000
_H = 128
_G = 64

_NC = 2
_NS = 16
_NP = 10240
_RPT = _NP // _NS
_K = 128
_CHUNKS = 79
_EPT = _K * _CHUNKS
_EPC = _EPT * _NS
_EP = _EPC * _NC
_DST_PAD = _N + 16

_mesh = plsc.VectorSubcoreMesh(core_axis_name="c", subcore_axis_name="s")


@functools.partial(
    pl.kernel,
    out_type=jax.ShapeDtypeStruct((_NC * _NP, 16), jnp.float32),
    mesh=_mesh,
    scratch_types=[
        pltpu.VMEM((_K,), jnp.int32),
        pltpu.VMEM((_K, 16), jnp.float32),
        pltpu.VMEM_SHARED((_NP, 16), jnp.float32),
    ],
)
def _sc_deg(dst_hbm, ones_hbm, zeros_hbm, out_hbm, didx, ones_v, acc):
    cid = lax.axis_index("c")
    sid = lax.axis_index("s")
    pltpu.sync_copy(ones_hbm, ones_v)
    pltpu.sync_copy(zeros_hbm, acc.at[pl.ds(sid * _RPT, _RPT)])
    plsc.subcore_barrier()
    base = cid * _EPC + sid * _EPT

    def body(c, _):
        pltpu.sync_copy(dst_hbm.at[pl.ds(base + c * _K, _K)], didx)
        pltpu.sync_copy(ones_v, acc.at[didx], add=True)
        return 0

    lax.fori_loop(0, _CHUNKS, body, 0)
    plsc.subcore_barrier()
    pltpu.sync_copy(
        acc.at[pl.ds(sid * _RPT, _RPT)],
        out_hbm.at[pl.ds(cid * _NP + sid * _RPT, _RPT)],
    )


@functools.partial(
    pl.kernel,
    out_type=jax.ShapeDtypeStruct((_NC * _NP, _H), jnp.float32),
    mesh=_mesh,
    scratch_types=[
        pltpu.VMEM((_K,), jnp.int32),
        pltpu.VMEM((_K,), jnp.int32),
        pltpu.VMEM((_K, _H), jnp.float32),
        pltpu.VMEM_SHARED((_NP, _H), jnp.float32),
        pltpu.SemaphoreType.DMA,
    ],
)
def _sc_prop(src_hbm, dst_hbm, hs_hbm, zeros_hbm, out_hbm, sidx, didx, rows, acc, sem):
    cid = lax.axis_index("c")
    sid = lax.axis_index("s")
    pltpu.sync_copy(zeros_hbm, acc.at[pl.ds(sid * _RPT, _RPT)])
    plsc.subcore_barrier()
    base = cid * _EPC + sid * _EPT

    def body(c, _):
        off = base + c * _K
        pltpu.sync_copy(src_hbm.at[pl.ds(off, _K)], sidx)
        pltpu.sync_copy(dst_hbm.at[pl.ds(off, _K)], didx)
        pltpu.async_copy(hs_hbm.at[sidx], rows, sem).wait()
        pltpu.sync_copy(rows, acc.at[didx], add=True)
        return 0

    lax.fori_loop(0, _CHUNKS, body, 0)
    plsc.subcore_barrier()
    pltpu.sync_copy(
        acc.at[pl.ds(sid * _RPT, _RPT)],
        out_hbm.at[pl.ds(cid * _NP + sid * _RPT, _RPT)],
    )


_BLK = 512
_GRID = _NP // _BLK


def _pre_body(xr, wr, da, db, dis_r, p_r, hs_r):
    deg = da[:, :1] + db[:, :1] + 1.0
    dis = lax.rsqrt(deg)
    p = jnp.dot(xr[...], wr[...], preferred_element_type=jnp.float32)
    dis_r[...] = dis
    p_r[...] = p
    hs_r[...] = p * dis


def _tc_pre(x_p, W1, degp):
    return pl.pallas_call(
        _pre_body,
        grid=(_GRID,),
        in_specs=[
            pl.BlockSpec((_BLK, _H), lambda i: (i, 0)),
            pl.BlockSpec((_H, _H), lambda i: (0, 0)),
            pl.BlockSpec((_BLK, 16), lambda i: (i, 0)),
            pl.BlockSpec((_BLK, 16), lambda i: (i + _GRID, 0)),
        ],
        out_specs=[
            pl.BlockSpec((_BLK, 1), lambda i: (i, 0)),
            pl.BlockSpec((_BLK, _H), lambda i: (i, 0)),
            pl.BlockSpec((_BLK, _H), lambda i: (i, 0)),
        ],
        out_shape=[
            jax.ShapeDtypeStruct((_NP, 1), jnp.float32),
            jax.ShapeDtypeStruct((_NP, _H), jnp.float32),
            jax.ShapeDtypeStruct((_NP, _H), jnp.float32),
        ],
    )(x_p, W1, degp, degp)


def _mid_body(sa, sb, pr, disr, br, wr, pn_r, hsn_r):
    dis = disr[...]
    h = (sa[...] + sb[...]) * dis + pr[...] * (dis * dis) + br[...]
    h = jnp.maximum(h, 0.0)
    pn = jnp.dot(h, wr[...], preferred_element_type=jnp.float32)
    pn_r[...] = pn
    hsn_r[...] = pn * dis


def _tc_mid(s_parts, p, dis, b, W_next):
    return pl.pallas_call(
        _mid_body,
        grid=(_GRID,),
        in_specs=[
            pl.BlockSpec((_BLK, _H), lambda i: (i, 0)),
            pl.BlockSpec((_BLK, _H), lambda i: (i + _GRID, 0)),
            pl.BlockSpec((_BLK, _H), lambda i: (i, 0)),
            pl.BlockSpec((_BLK, 1), lambda i: (i, 0)),
            pl.BlockSpec((1, _H), lambda i: (0, 0)),
            pl.BlockSpec((_H, _H), lambda i: (0, 0)),
        ],
        out_specs=[
            pl.BlockSpec((_BLK, _H), lambda i: (i, 0)),
            pl.BlockSpec((_BLK, _H), lambda i: (i, 0)),
        ],
        out_shape=[
            jax.ShapeDtypeStruct((_NP, _H), jnp.float32),
            jax.ShapeDtypeStruct((_NP, _H), jnp.float32),
        ],
    )(s_parts, s_parts, p, dis, b, W_next)


def _final_body(sa, sb, pr, disr, br, batchr, lwr, lbr, out_r, sums, counts):
    i = pl.program_id(0)

    @pl.when(i == 0)
    def _():
        sums[...] = jnp.zeros_like(sums)
        counts[...] = jnp.zeros_like(counts)

    dis = disr[...]
    h = (sa[...] + sb[...]) * dis + pr[...] * (dis * dis) + br[...]
    gids = lax.broadcasted_iota(jnp.int32, (_BLK, _G), 1)
    m = (batchr[...] == gids).astype(jnp.float32)
    sums[...] += lax.dot_general(
        m, h, (((0,), (0,)), ((), ())), preferred_element_type=jnp.float32
    )
    counts[...] += lax.dot_general(
        m,
        jnp.ones((_BLK, 1), jnp.float32),
        (((0,), (0,)), ((), ())),
        preferred_element_type=jnp.float32,
    )

    @pl.when(i == _GRID - 1)
    def _():
        pooled = sums[...] / jnp.maximum(counts[...], 1.0)
        out_r[...] = (
            jnp.dot(pooled, lwr[...], preferred_element_type=jnp.float32) + lbr[...]
        )


def _tc_final(s_parts, p, dis, b, batch2d, lin_W, lin_b2):
    return pl.pallas_call(
        _final_body,
        grid=(_GRID,),
        in_specs=[
            pl.BlockSpec((_BLK, _H), lambda i: (i, 0)),
            pl.BlockSpec((_BLK, _H), lambda i: (i + _GRID, 0)),
            pl.BlockSpec((_BLK, _H), lambda i: (i, 0)),
            pl.BlockSpec((_BLK, 1), lambda i: (i, 0)),
            pl.BlockSpec((1, _H), lambda i: (0, 0)),
            pl.BlockSpec((_BLK, 1), lambda i: (i, 0)),
            pl.BlockSpec((_H, 1), lambda i: (0, 0)),
            pl.BlockSpec((1, 1), lambda i: (0, 0)),
        ],
        out_specs=pl.BlockSpec((_G, 1), lambda i: (0, 0)),
        out_shape=jax.ShapeDtypeStruct((_G, 1), jnp.float32),
        scratch_shapes=[
            pltpu.VMEM((_G, _H), jnp.float32),
            pltpu.VMEM((_G, 1), jnp.float32),
        ],
    )(s_parts, s_parts, p, dis, b, batch2d, lin_W, lin_b2)


def kernel(x, edge_index, batch, W1, b1, W2, b2, W3, b3, lin_W, lin_b):
    src = edge_index[0]
    dst = edge_index[1]
    src_p = jnp.concatenate([src, jnp.zeros((_EP - _E,), jnp.int32)])
    dst_p = jnp.concatenate(
        [dst, jnp.full((_EP - _E,), _DST_PAD, jnp.int32)]
    )
    x_p = jnp.pad(x, ((0, _NP - _N), (0, 0)))
    batch2d = jnp.pad(batch, (0, _NP - _N), constant_values=_G).reshape(_NP, 1)

    zeros_h = jnp.zeros((_RPT, _H), jnp.float32)
    zeros_16 = jnp.zeros((_RPT, 16), jnp.float32)
    ones_16 = jnp.ones((_K, 16), jnp.float32)

    degp = _sc_deg(dst_p, ones_16, zeros_16)
    dis, p1, hs1 = _tc_pre(x_p, W1, degp)
    s1 = _sc_prop(src_p, dst_p, hs1, zeros_h)
    p2, hs2 = _tc_mid(s1, p1, dis, b1.reshape(1, _H), W2)
    s2 = _sc_prop(src_p, dst_p, hs2, zeros_h)
    p3, hs3 = _tc_mid(s2, p2, dis, b2.reshape(1, _H), W3)
    s3 = _sc_prop(src_p, dst_p, hs3, zeros_h)
    return _tc_final(
        s3, p3, dis, b3.reshape(1, _H), batch2d, lin_W, lin_b.reshape(1, 1)
    )
